# baseline (speedup 1.0000x reference)
import functools

import numpy as np
import jax
import jax.numpy as jnp
from jax import lax
from jax.experimental import pallas as pl
from jax.experimental.pallas import tpu as pltpu
from jax.experimental.pallas import tpu_sc as plsc

VOCAB = 100000
EMBD = 128
MAX_LEN = 8192
BATCH = 4
SEQ = 8192
TOK = BATCH * SEQ
NW = 32
SRANGE = SEQ // NW
PER = BATCH * SRANGE
QCHUNK = 64
NGROUP = SRANGE // QCHUNK
LANES = 16
NSLOT = 2 * BATCH


def _position_table() -> np.ndarray:
    pos = np.arange(MAX_LEN, dtype=np.float64)[:, None]
    div = np.arange(0, EMBD, 2, dtype=np.float64)[None, :]
    m = (pos / (10000.0 ** (div / EMBD))).astype(np.float32)
    return np.concatenate([np.sin(m), np.cos(m)], axis=-1).astype(np.float32)


_POS_FLAT = _position_table().reshape(-1)

_MESH = plsc.VectorSubcoreMesh(core_axis_name="c", subcore_axis_name="s")


@functools.partial(
    pl.kernel,
    mesh=_MESH,
    out_type=[
        jax.ShapeDtypeStruct((BATCH, SEQ, EMBD), jnp.float32),
        jax.ShapeDtypeStruct((NW, BATCH, LANES), jnp.int32),
    ],
    scratch_types=[
        pltpu.VMEM((BATCH, SRANGE), jnp.int32),
        pltpu.VMEM((SRANGE * EMBD,), jnp.float32),
        pltpu.VMEM((NSLOT, QCHUNK, EMBD), jnp.float32),
        pltpu.VMEM((BATCH, LANES), jnp.int32),
        pltpu.SemaphoreType.DMA,
        pltpu.SemaphoreType.DMA((NSLOT,)),
        pltpu.SemaphoreType.DMA((NSLOT,)),
    ],
)
def _embed_sc(ids_h, tab_h, pos_h, out_h, cnt_h,
              idx_v, pbuf, gring, cnt_v, psem, gsar, osar):
    wid = lax.axis_index("s") * 2 + lax.axis_index("c")
    sbase = wid * SRANGE

    pcp = pltpu.async_copy(pos_h.at[pl.ds(sbase * EMBD, SRANGE * EMBD)],
                           pbuf, psem)

    pltpu.sync_copy(ids_h.at[:, pl.ds(sbase, SRANGE)], idx_v)

    def slot_of(g, b):
        return (g % 2) * BATCH + b

    def issue(g, b):
        s = slot_of(g, b)
        return pltpu.async_copy(
            tab_h.at[idx_v.at[b, pl.ds(g * QCHUNK, QCHUNK)]],
            gring.at[s], gsar.at[s])

    gathers = {}
    ostores = {}
    for b in range(BATCH):
        gathers[(0, b)] = issue(0, b)
    for b in range(BATCH):
        gathers[(1, b)] = issue(1, b)

    for b in range(BATCH):
        def count_body(t, acc):
            v = idx_v[b, pl.ds(t * LANES, LANES)]
            return acc + jnp.where(v == 1, 1, 0).astype(jnp.int32)

        acc = lax.fori_loop(0, SRANGE // LANES, count_body,
                            jnp.zeros((LANES,), jnp.int32))
        cnt_v[b] = acc
    pltpu.sync_copy(cnt_v, cnt_h.at[wid])

    pcp.wait()

    for g in range(NGROUP):
        for b in range(BATCH):
            gathers[(g, b)].wait()
        slots = [slot_of(g, b) for b in range(BATCH)]

        def add_rows(r0):
            def add_body(rr, _):
                r = r0 + rr
                pb = (g * QCHUNK + r) * EMBD

                for j in range(EMBD // LANES):
                    pv = pbuf[pl.ds(pb + j * LANES, LANES)]
                    for s in slots:
                        plsc.addupdate(
                            gring.at[s, r, pl.ds(j * LANES, LANES)], pv)
                return 0

            lax.fori_loop(0, QCHUNK // 2, add_body, 0)

        add_rows(0)
        if 1 <= g < NGROUP - 1:
            for b in range(BATCH):
                ostores[(g - 1, b)].wait()
                gathers[(g + 1, b)] = issue(g + 1, b)
        add_rows(QCHUNK // 2)

        for b in range(BATCH):
            ostores[(g, b)] = pltpu.async_copy(
                gring.at[slot_of(g, b)],
                out_h.at[b, pl.ds(sbase + g * QCHUNK, QCHUNK)],
                osar.at[slot_of(g, b)])

    for b in range(BATCH):
        ostores[(NGROUP - 2, b)].wait()
        ostores[(NGROUP - 1, b)].wait()


def kernel(ids, word_embedding):
    pos = jnp.asarray(_POS_FLAT)
    out, cnt = _embed_sc(ids, word_embedding, pos)
    padding_len = jnp.sum(cnt, axis=(0, 2))
    return (out, padding_len)
